# baseline (speedup 1.0000x reference)
import functools

import jax
import jax.numpy as jnp
from jax import lax
from jax.experimental import pallas as pl
from jax.experimental.pallas import tpu as pltpu
from jax.experimental.pallas import tpu_sc as plsc

_HID = 128
_NEI = 32
_G = 4
_L = 16
_NSUB = 16
_NG0 = 128
_NG1 = 36
_NBUF = 4


@functools.lru_cache(maxsize=None)
def _build(n_pad, n_nodes):
  pair_rows = (_NG0 + _NG1) * _G
  assert n_pad == _NSUB * pair_rows
  mesh = plsc.VectorSubcoreMesh(core_axis_name="c", subcore_axis_name="s",
                                num_cores=2, num_subcores=16)

  def body(nei_hbm, h_hbm, out_hbm, idx0, idx1, idx2, idx3,
           rows0, rows1, rows2, rows3, out_v, sem0, sem1, sem2, sem3):
    bufs = ((idx0, rows0, sem0), (idx1, rows1, sem1),
            (idx2, rows2, sem2), (idx3, rows3, sem3))
    c = lax.axis_index("c")
    s = lax.axis_index("s")
    row_base = s * pair_rows + jnp.where(c == 0, 0, _NG0 * _G)
    ng = jnp.where(c == 0, _NG0, _NG1)

    def start(g, idx_v, rows_v, sem):
      ibase = (row_base + g * _G) * _NEI
      pltpu.sync_copy(nei_hbm.at[pl.ds(ibase, _G * _NEI)], idx_v)
      return pltpu.async_copy(h_hbm.at[idx_v], rows_v, sem)

    def reduce_group(g, rows_v):
      def jstep(j, accs):
        new = []
        for d in range(_G):
          r = d * _NEI + j
          for k in range(_HID // _L):
            new.append(accs[d * (_HID // _L) + k]
                       + rows_v[r, pl.ds(k * _L, _L)])
        return tuple(new)

      init = tuple(jnp.zeros((_L,), jnp.float32)
                   for _ in range(_G * (_HID // _L)))
      accs = lax.fori_loop(0, _NEI, jstep, init)
      inv = jnp.float32(1.0 / _NEI)
      for d in range(_G):
        for k in range(_HID // _L):
          out_v[d, pl.ds(k * _L, _L)] = accs[d * (_HID // _L) + k] * inv
      pltpu.sync_copy(out_v, out_hbm.at[pl.ds(row_base + g * _G, _G)])

    for b in range(_NBUF):
      start(b, *bufs[b])

    def outer(i, _):
      g = i * _NBUF
      for b in range(_NBUF):
        gb = g + b
        idx_v, rows_v, sem = bufs[b]
        pltpu.make_async_copy(h_hbm.at[idx_v], rows_v, sem).wait()
        reduce_group(gb, rows_v)

        @pl.when(gb + _NBUF < ng)
        def _():
          start(gb + _NBUF, idx_v, rows_v, sem)
      return 0

    lax.fori_loop(0, ng // _NBUF, outer, 0)

  grid_kernel = pl.kernel(
      body,
      out_type=jax.ShapeDtypeStruct((n_pad, _HID), jnp.float32),
      mesh=mesh,
      scratch_types=(
          [pltpu.VMEM((_G * _NEI,), jnp.int32)] * _NBUF
          + [pltpu.VMEM((_G * _NEI, _HID), jnp.float32)] * _NBUF
          + [pltpu.VMEM((_G, _HID), jnp.float32)]
          + [pltpu.SemaphoreType.DMA] * _NBUF
      ),
  )
  return grid_kernel


@jax.jit
def kernel(nei, h, h_refer, att):
  n_dst = nei.shape[0]
  n_pad = _NSUB * (_NG0 + _NG1) * _G
  nei_flat = jnp.pad(nei.astype(jnp.int32),
                     ((0, n_pad - n_dst), (0, 0))).reshape(-1)
  out = _build(n_pad, h.shape[0])(nei_flat, h)
  return out[:n_dst]


# speedup vs baseline: 2.1083x; 2.1083x over previous
import functools

import jax
import jax.numpy as jnp
from jax import lax
from jax.experimental import pallas as pl
from jax.experimental.pallas import tpu as pltpu
from jax.experimental.pallas import tpu_sc as plsc

_HID = 128
_NEI = 32
_G = 4
_L = 16
_NSUB = 16
_NG0 = 124
_NG1 = 36
_NBUF = 4


@functools.lru_cache(maxsize=None)
def _build(n_pad, n_nodes):
  pair_rows = (_NG0 + _NG1) * _G
  assert n_pad == _NSUB * pair_rows
  mesh = plsc.VectorSubcoreMesh(core_axis_name="c", subcore_axis_name="s",
                                num_cores=2, num_subcores=16)

  def body(nei_hbm, h_hbm, out_hbm, idx0, idx1, idx2, idx3,
           rows0, rows1, rows2, rows3, out_v, sem0, sem1, sem2, sem3):
    bufs = ((idx0, rows0, sem0), (idx1, rows1, sem1),
            (idx2, rows2, sem2), (idx3, rows3, sem3))
    c = lax.axis_index("c")
    s = lax.axis_index("s")
    row_base = s * pair_rows + jnp.where(c == 0, 0, _NG0 * _G)
    ng = jnp.where(c == 0, _NG0, _NG1)

    def start(g, idx_v, rows_v, sem):
      ibase = (row_base + g * _G) * _NEI
      pltpu.sync_copy(nei_hbm.at[pl.ds(ibase, _G * _NEI)], idx_v)
      return pltpu.async_copy(h_hbm.at[idx_v], rows_v, sem)

    def reduce_group(g, rows_v):
      def jstep(j, accs):
        new = []
        for d in range(_G):
          r = d * _NEI + j
          for k in range(_HID // _L):
            new.append(accs[d * (_HID // _L) + k]
                       + rows_v[r, pl.ds(k * _L, _L)])
        return tuple(new)

      init = tuple(jnp.zeros((_L,), jnp.float32)
                   for _ in range(_G * (_HID // _L)))
      accs = lax.fori_loop(0, _NEI, jstep, init)
      inv = jnp.float32(1.0 / _NEI)
      for d in range(_G):
        for k in range(_HID // _L):
          out_v[d, pl.ds(k * _L, _L)] = accs[d * (_HID // _L) + k] * inv
      pltpu.sync_copy(out_v, out_hbm.at[pl.ds(row_base + g * _G, _G)])

    for b in range(_NBUF):
      start(b, *bufs[b])

    def outer(i, _):
      g = i * _NBUF
      for b in range(_NBUF):
        gb = g + b
        idx_v, rows_v, sem = bufs[b]
        pltpu.make_async_copy(h_hbm.at[idx_v], rows_v, sem).wait()
        reduce_group(gb, rows_v)

        @pl.when(gb + _NBUF < ng)
        def _():
          start(gb + _NBUF, idx_v, rows_v, sem)
      return 0

    lax.fori_loop(0, ng // _NBUF, outer, 0)

  grid_kernel = pl.kernel(
      body,
      out_type=jax.ShapeDtypeStruct((n_pad, _HID), jnp.float32),
      mesh=mesh,
      scratch_types=(
          [pltpu.VMEM((_G * _NEI,), jnp.int32)] * _NBUF
          + [pltpu.VMEM((_G * _NEI, _HID), jnp.float32)] * _NBUF
          + [pltpu.VMEM((_G, _HID), jnp.float32)]
          + [pltpu.SemaphoreType.DMA] * _NBUF
      ),
  )
  return grid_kernel


@jax.jit
def kernel(nei, h, h_refer, att):
  n_dst = nei.shape[0]
  n_pad = _NSUB * (_NG0 + _NG1) * _G
  nei_flat = jnp.pad(nei.astype(jnp.int32),
                     ((0, n_pad - n_dst), (0, 0))).reshape(-1)
  out = _build(n_pad, h.shape[0])(nei_flat, h)
  return out[:n_dst]


# speedup vs baseline: 3.2785x; 1.5551x over previous
import functools

import jax
import jax.numpy as jnp
from jax import lax
from jax.experimental import pallas as pl
from jax.experimental.pallas import tpu as pltpu
from jax.experimental.pallas import tpu_sc as plsc

_HID = 128
_NEI = 32
_G = 4
_L = 16
_NSUB = 16
_NCHUNK = _HID // (2 * _L)
_NG0 = 124
_NG1 = 34
_NBUF = 2


@functools.lru_cache(maxsize=None)
def _build(n_pad, n_nodes):
  pair_rows = (_NG0 + _NG1) * _G
  assert n_pad == _NSUB * pair_rows
  mesh = plsc.VectorSubcoreMesh(core_axis_name="c", subcore_axis_name="s",
                                num_cores=2, num_subcores=16)

  def body(nei_hbm, h_hbm, out_hbm, idx_all, rows0, rows1, ov0, ov1,
           gsem0, gsem1, osem0, osem1):
    bufs = ((rows0, ov0, gsem0, osem0), (rows1, ov1, gsem1, osem1))
    c = lax.axis_index("c")
    s = lax.axis_index("s")
    row_base = s * pair_rows + jnp.where(c == 0, 0, _NG0 * _G)
    ng = jnp.where(c == 0, _NG0, _NG1)
    ibase = row_base * _NEI

    @pl.when(c == 0)
    def _():
      n = _NG0 * _G * _NEI
      pltpu.sync_copy(nei_hbm.at[pl.ds(ibase, n)], idx_all.at[pl.ds(0, n)])

    @pl.when(c == 1)
    def _():
      n = _NG1 * _G * _NEI
      pltpu.sync_copy(nei_hbm.at[pl.ds(ibase, n)], idx_all.at[pl.ds(0, n)])

    def start(g, rows_v, gsem):
      idx = idx_all.at[pl.ds(g * _G * _NEI, _G * _NEI)]
      return pltpu.async_copy(h_hbm.at[idx], rows_v, gsem)

    def reduce_group(g, rows_v, ov, osem):
      @pl.when(g >= _NBUF)
      def _():
        pltpu.make_async_copy(
            ov, out_hbm.at[pl.ds(row_base + g * _G, _G)], osem).wait()

      def jstep(j, accs):
        new = []
        for d in range(_G):
          r = d * _NEI + j
          for k in range(_HID // _L):
            new.append(accs[d * (_HID // _L) + k]
                       + rows_v[r, pl.ds(k * _L, _L)])
        return tuple(new)

      init = tuple(jnp.zeros((_L,), jnp.float32)
                   for _ in range(_G * (_HID // _L)))
      accs = lax.fori_loop(0, _NEI, jstep, init)
      inv = jnp.float32(1.0 / _NEI)
      for d in range(_G):
        for k in range(_HID // _L):
          ov[d, pl.ds(k * _L, _L)] = accs[d * (_HID // _L) + k] * inv
      pltpu.async_copy(ov, out_hbm.at[pl.ds(row_base + g * _G, _G)], osem)

    for b in range(_NBUF):
      start(b, bufs[b][0], bufs[b][2])

    def outer(i, _):
      g = i * _NBUF
      for b in range(_NBUF):
        gb = g + b
        rows_v, ov, gsem, osem = bufs[b]
        pltpu.make_async_copy(h_hbm.at[idx_all.at[pl.ds(0, _G * _NEI)]],
                              rows_v, gsem).wait()
        reduce_group(gb, rows_v, ov, osem)

        @pl.when(gb + _NBUF < ng)
        def _():
          start(gb + _NBUF, rows_v, gsem)
      return 0

    lax.fori_loop(0, ng // _NBUF, outer, 0)

    for b in range(_NBUF):
      rows_v, ov, gsem, osem = bufs[b]
      pltpu.make_async_copy(
          ov, out_hbm.at[pl.ds(row_base, _G)], osem).wait()

  grid_kernel = pl.kernel(
      body,
      out_type=jax.ShapeDtypeStruct((n_pad, _HID), jnp.float32),
      mesh=mesh,
      scratch_types=(
          [pltpu.VMEM((_NG0 * _G * _NEI,), jnp.int32)]
          + [pltpu.VMEM((_G * _NEI, _HID), jnp.float32)] * _NBUF
          + [pltpu.VMEM((_G, _HID), jnp.float32)] * _NBUF
          + [pltpu.SemaphoreType.DMA] * (2 * _NBUF)
      ),
  )
  return grid_kernel


@jax.jit
def kernel(nei, h, h_refer, att):
  n_dst = nei.shape[0]
  n_pad = _NSUB * (_NG0 + _NG1) * _G
  nei_flat = jnp.pad(nei.astype(jnp.int32),
                     ((0, n_pad - n_dst), (0, 0))).reshape(-1)
  out = _build(n_pad, h.shape[0])(nei_flat, h)
  return out[:n_dst]


# speedup vs baseline: 3.4059x; 1.0389x over previous
import functools

import jax
import jax.numpy as jnp
from jax import lax
from jax.experimental import pallas as pl
from jax.experimental.pallas import tpu as pltpu
from jax.experimental.pallas import tpu_sc as plsc

_HID = 128
_NEI = 32
_G = 4
_L = 16
_NSUB = 16
_NCHUNK = _HID // (2 * _L)
_NG0 = 132
_NG1 = 26
_NBUF = 2


@functools.lru_cache(maxsize=None)
def _build(n_pad, n_nodes):
  pair_rows = (_NG0 + _NG1) * _G
  assert n_pad == _NSUB * pair_rows
  mesh = plsc.VectorSubcoreMesh(core_axis_name="c", subcore_axis_name="s",
                                num_cores=2, num_subcores=16)

  def body(nei_hbm, h_hbm, out_hbm, idx_all, rows0, rows1, ov0, ov1,
           gsem0, gsem1, osem0, osem1):
    bufs = ((rows0, ov0, gsem0, osem0), (rows1, ov1, gsem1, osem1))
    c = lax.axis_index("c")
    s = lax.axis_index("s")
    row_base = s * pair_rows + jnp.where(c == 0, 0, _NG0 * _G)
    ng = jnp.where(c == 0, _NG0, _NG1)
    ibase = row_base * _NEI

    @pl.when(c == 0)
    def _():
      n = _NG0 * _G * _NEI
      pltpu.sync_copy(nei_hbm.at[pl.ds(ibase, n)], idx_all.at[pl.ds(0, n)])

    @pl.when(c == 1)
    def _():
      n = _NG1 * _G * _NEI
      pltpu.sync_copy(nei_hbm.at[pl.ds(ibase, n)], idx_all.at[pl.ds(0, n)])

    def start(g, rows_v, gsem):
      idx = idx_all.at[pl.ds(g * _G * _NEI, _G * _NEI)]
      return pltpu.async_copy(h_hbm.at[idx], rows_v, gsem)

    def reduce_group(g, rows_v, ov, osem):
      @pl.when(g >= _NBUF)
      def _():
        pltpu.make_async_copy(
            ov, out_hbm.at[pl.ds(row_base + g * _G, _G)], osem).wait()

      def jstep(j, accs):
        new = []
        for d in range(_G):
          r = d * _NEI + j
          for k in range(_HID // _L):
            new.append(accs[d * (_HID // _L) + k]
                       + rows_v[r, pl.ds(k * _L, _L)])
        return tuple(new)

      init = tuple(jnp.zeros((_L,), jnp.float32)
                   for _ in range(_G * (_HID // _L)))
      accs = lax.fori_loop(0, _NEI, jstep, init)
      inv = jnp.float32(1.0 / _NEI)
      for d in range(_G):
        for k in range(_HID // _L):
          ov[d, pl.ds(k * _L, _L)] = accs[d * (_HID // _L) + k] * inv
      pltpu.async_copy(ov, out_hbm.at[pl.ds(row_base + g * _G, _G)], osem)

    for b in range(_NBUF):
      start(b, bufs[b][0], bufs[b][2])

    def outer(i, _):
      g = i * _NBUF
      for b in range(_NBUF):
        gb = g + b
        rows_v, ov, gsem, osem = bufs[b]
        pltpu.make_async_copy(h_hbm.at[idx_all.at[pl.ds(0, _G * _NEI)]],
                              rows_v, gsem).wait()
        reduce_group(gb, rows_v, ov, osem)

        @pl.when(gb + _NBUF < ng)
        def _():
          start(gb + _NBUF, rows_v, gsem)
      return 0

    lax.fori_loop(0, ng // _NBUF, outer, 0)

    for b in range(_NBUF):
      rows_v, ov, gsem, osem = bufs[b]
      pltpu.make_async_copy(
          ov, out_hbm.at[pl.ds(row_base, _G)], osem).wait()

  grid_kernel = pl.kernel(
      body,
      out_type=jax.ShapeDtypeStruct((n_pad, _HID), jnp.float32),
      mesh=mesh,
      scratch_types=(
          [pltpu.VMEM((_NG0 * _G * _NEI,), jnp.int32)]
          + [pltpu.VMEM((_G * _NEI, _HID), jnp.float32)] * _NBUF
          + [pltpu.VMEM((_G, _HID), jnp.float32)] * _NBUF
          + [pltpu.SemaphoreType.DMA] * (2 * _NBUF)
      ),
  )
  return grid_kernel


@jax.jit
def kernel(nei, h, h_refer, att):
  n_dst = nei.shape[0]
  n_pad = _NSUB * (_NG0 + _NG1) * _G
  nei_flat = jnp.pad(nei.astype(jnp.int32),
                     ((0, n_pad - n_dst), (0, 0))).reshape(-1)
  out = _build(n_pad, h.shape[0])(nei_flat, h)
  return out[:n_dst]


# speedup vs baseline: 3.4137x; 1.0023x over previous
import functools

import jax
import jax.numpy as jnp
from jax import lax
from jax.experimental import pallas as pl
from jax.experimental.pallas import tpu as pltpu
from jax.experimental.pallas import tpu_sc as plsc

_HID = 128
_NEI = 32
_G = 4
_L = 16
_NSUB = 16
_NCHUNK = _HID // (2 * _L)
_NG0 = 132
_NG1 = 26
_NBUF = 2


@functools.lru_cache(maxsize=None)
def _build(n_pad, n_nodes):
  pair_rows = (_NG0 + _NG1) * _G
  assert n_pad == _NSUB * pair_rows
  mesh = plsc.VectorSubcoreMesh(core_axis_name="c", subcore_axis_name="s",
                                num_cores=2, num_subcores=16)

  def body(nei_hbm, h_hbm, out_hbm, idx_all, rows0, rows1, ov0, ov1,
           gsem0, gsem1, osem0, osem1):
    bufs = ((rows0, ov0, gsem0, osem0), (rows1, ov1, gsem1, osem1))
    c = lax.axis_index("c")
    s = lax.axis_index("s")
    row_base = s * pair_rows + jnp.where(c == 0, 0, _NG0 * _G)
    ng = jnp.where(c == 0, _NG0, _NG1)
    ibase = row_base * _NEI

    @pl.when(c == 0)
    def _():
      n = _NG0 * _G * _NEI
      pltpu.sync_copy(nei_hbm.at[pl.ds(ibase, n)], idx_all.at[pl.ds(0, n)])

    @pl.when(c == 1)
    def _():
      n = _NG1 * _G * _NEI
      pltpu.sync_copy(nei_hbm.at[pl.ds(ibase, n)], idx_all.at[pl.ds(0, n)])

    def start(g, rows_v, gsem):
      idx = idx_all.at[pl.ds(g * _G * _NEI, _G * _NEI)]
      return pltpu.async_copy(h_hbm.at[idx], rows_v, gsem)

    def reduce_group(g, rows_v, ov, osem):
      @pl.when(g >= _NBUF)
      def _():
        pltpu.make_async_copy(
            ov, out_hbm.at[pl.ds(row_base + g * _G, _G)], osem).wait()

      def jstep(j2, accs):
        for u in range(2):
          j = j2 * 2 + u
          new = []
          for d in range(_G):
            r = d * _NEI + j
            for k in range(_HID // _L):
              new.append(accs[d * (_HID // _L) + k]
                         + rows_v[r, pl.ds(k * _L, _L)])
          accs = tuple(new)
        return accs

      init = tuple(jnp.zeros((_L,), jnp.float32)
                   for _ in range(_G * (_HID // _L)))
      accs = lax.fori_loop(0, _NEI // 2, jstep, init)
      inv = jnp.float32(1.0 / _NEI)
      for d in range(_G):
        for k in range(_HID // _L):
          ov[d, pl.ds(k * _L, _L)] = accs[d * (_HID // _L) + k] * inv
      pltpu.async_copy(ov, out_hbm.at[pl.ds(row_base + g * _G, _G)], osem)

    for b in range(_NBUF):
      start(b, bufs[b][0], bufs[b][2])

    def outer(i, _):
      g = i * _NBUF
      for b in range(_NBUF):
        gb = g + b
        rows_v, ov, gsem, osem = bufs[b]
        pltpu.make_async_copy(h_hbm.at[idx_all.at[pl.ds(0, _G * _NEI)]],
                              rows_v, gsem).wait()
        reduce_group(gb, rows_v, ov, osem)

        @pl.when(gb + _NBUF < ng)
        def _():
          start(gb + _NBUF, rows_v, gsem)
      return 0

    lax.fori_loop(0, ng // _NBUF, outer, 0)

    for b in range(_NBUF):
      rows_v, ov, gsem, osem = bufs[b]
      pltpu.make_async_copy(
          ov, out_hbm.at[pl.ds(row_base, _G)], osem).wait()

  grid_kernel = pl.kernel(
      body,
      out_type=jax.ShapeDtypeStruct((n_pad, _HID), jnp.float32),
      mesh=mesh,
      scratch_types=(
          [pltpu.VMEM((_NG0 * _G * _NEI,), jnp.int32)]
          + [pltpu.VMEM((_G * _NEI, _HID), jnp.float32)] * _NBUF
          + [pltpu.VMEM((_G, _HID), jnp.float32)] * _NBUF
          + [pltpu.SemaphoreType.DMA] * (2 * _NBUF)
      ),
  )
  return grid_kernel


@jax.jit
def kernel(nei, h, h_refer, att):
  n_dst = nei.shape[0]
  n_pad = _NSUB * (_NG0 + _NG1) * _G
  nei_flat = jnp.pad(nei.astype(jnp.int32),
                     ((0, n_pad - n_dst), (0, 0))).reshape(-1)
  out = _build(n_pad, h.shape[0])(nei_flat, h)
  return out[:n_dst]
